# Initial kernel scaffold; baseline (speedup 1.0000x reference)
import jax
import jax.numpy as jnp
from jax.experimental import pallas as pl


def kernel(z, codebook):
    raise NotImplementedError("write your pallas kernel here")


# speedup vs baseline: 1.3351x; 1.3351x over previous
import functools

import jax
import jax.numpy as jnp
from jax import lax
from jax.experimental import pallas as pl
from jax.experimental.pallas import tpu as pltpu

N_E = 512
E_D = 32
R = 1024


def _vq_body(z_ref, cb_ref, cbt_ref, zq_ref, idx_ref, loss_ref):
    b = pl.program_id(0)
    n = pl.program_id(1)
    zblk = z_ref[0]
    cb = cb_ref[...]
    cbt = cbt_ref[...]

    zt = jnp.transpose(zblk)
    zn = jnp.sum(zt * zt, axis=1, keepdims=True)
    cn = jnp.sum(cbt * cbt, axis=0, keepdims=True)
    zc = lax.dot_general(
        zt.astype(jnp.bfloat16), cbt.astype(jnp.bfloat16),
        (((1,), (0,)), ((), ())),
        preferred_element_type=jnp.float32)
    dists = (zn + cn) - 2.0 * zc

    m = jnp.min(dists, axis=1, keepdims=True)
    ii = lax.broadcasted_iota(jnp.int32, (R, N_E), 1)
    idx = jnp.min(jnp.where(dists == m, ii, N_E), axis=1, keepdims=True)

    onehot = jnp.where(ii == idx, 1.0, 0.0)
    zq = lax.dot_general(
        cb, onehot, (((0,), (1,)), ((), ())),
        preferred_element_type=jnp.float32,
        precision=lax.Precision.HIGHEST)

    diff = zq - zblk
    zq_ref[0] = zblk + diff
    idx_ref[...] = idx.astype(jnp.float32).reshape(1, R, 1)

    @pl.when((b == 0) & (n == 0))
    def _():
        loss_ref[...] = jnp.zeros_like(loss_ref)

    loss_ref[...] += jnp.sum(diff * diff).reshape(1, 1)

    @pl.when((b == pl.num_programs(0) - 1) & (n == pl.num_programs(1) - 1))
    def _():
        loss_ref[...] = loss_ref[...] * (1.25 / (16 * 32 * 64 * 64))


@functools.partial(jax.jit, static_argnums=())
def _vq_tc(zr, codebook):
    B, C, HW = zr.shape
    nb = HW // R
    grid = (B, nb)
    return pl.pallas_call(
        _vq_body,
        grid=grid,
        in_specs=[
            pl.BlockSpec((1, C, R), lambda b, n: (b, 0, n)),
            pl.BlockSpec((N_E, E_D), lambda b, n: (0, 0)),
            pl.BlockSpec((E_D, N_E), lambda b, n: (0, 0)),
        ],
        out_specs=[
            pl.BlockSpec((1, C, R), lambda b, n: (b, 0, n)),
            pl.BlockSpec((1, R, 1), lambda b, n: (b * nb + n, 0, 0)),
            pl.BlockSpec((1, 1), lambda b, n: (0, 0)),
        ],
        out_shape=[
            jax.ShapeDtypeStruct((B, C, HW), jnp.float32),
            jax.ShapeDtypeStruct((B * nb, R, 1), jnp.float32),
            jax.ShapeDtypeStruct((1, 1), jnp.float32),
        ],
        compiler_params=pltpu.CompilerParams(
            dimension_semantics=("arbitrary", "arbitrary")),
    )(zr, codebook, codebook.T)


def kernel(z, codebook):
    B, C, H, W = z.shape
    zr = z.reshape(B, C, H * W)
    zq, idxf, loss = _vq_tc(zr, codebook)
    idxi = idxf.reshape(-1).astype(jnp.int32)
    usage = (jnp.bincount(idxi, length=N_E) > 0).sum().astype(jnp.float32) / float(N_E)
    return (zq.reshape(B, C, H, W), loss[0, 0], usage)


# speedup vs baseline: 1.5258x; 1.1428x over previous
import functools

import jax
import jax.numpy as jnp
from jax import lax
from jax.experimental import pallas as pl
from jax.experimental.pallas import tpu as pltpu

N_E = 512
E_D = 32
R = 1024


def _vq_body(z_ref, cbt_ref, cbtbf_ref, cb_ref, cn_ref,
             zq_ref, loss_ref, usage_ref, flags_ref):
    b = pl.program_id(0)
    n = pl.program_id(1)
    first = (b == 0) & (n == 0)
    last = ((b == pl.num_programs(0) - 1) & (n == pl.num_programs(1) - 1))

    zblk = z_ref[0]
    cb = cb_ref[...]
    cn = cn_ref[...]

    zt = jnp.transpose(zblk)
    zn = jnp.sum(zt * zt, axis=1, keepdims=True)
    zc = lax.dot_general(
        zt.astype(jnp.bfloat16), cbtbf_ref[...],
        (((1,), (0,)), ((), ())),
        preferred_element_type=jnp.float32)
    dists = (zn + cn) - 2.0 * zc

    m = jnp.min(dists, axis=1, keepdims=True)
    ii = lax.broadcasted_iota(jnp.int32, (1, N_E), 1)
    idx = jnp.min(jnp.where(dists == m, ii, N_E), axis=1, keepdims=True)

    onehot = jnp.where(ii == idx, 1.0, 0.0)
    zq = lax.dot_general(
        cb, onehot, (((0,), (1,)), ((), ())),
        preferred_element_type=jnp.float32,
        precision=lax.Precision.HIGHEST)

    diff = zq - zblk
    zq_ref[0] = zblk + diff

    fl_part = jnp.max(onehot, axis=0, keepdims=True)

    @pl.when(first)
    def _():
        loss_ref[...] = jnp.zeros_like(loss_ref)
        flags_ref[...] = fl_part

    @pl.when(jnp.logical_not(first))
    def _():
        flags_ref[...] = jnp.maximum(flags_ref[...], fl_part)

    loss_ref[...] += jnp.sum(diff * diff).reshape(1, 1)

    @pl.when(last)
    def _():
        loss_ref[...] = loss_ref[...] * (1.25 / (16 * 32 * 64 * 64))
        usage_ref[...] = (jnp.sum(flags_ref[...]) * (1.0 / N_E)).reshape(1, 1)


@jax.jit
def _vq_tc(zr, codebook):
    B, C, HW = zr.shape
    nb = HW // R
    grid = (B, nb)
    cbt = codebook.T
    cn = jnp.sum(codebook * codebook, axis=1)[None, :]
    return pl.pallas_call(
        _vq_body,
        grid=grid,
        in_specs=[
            pl.BlockSpec((1, C, R), lambda b, n: (b, 0, n)),
            pl.BlockSpec((E_D, N_E), lambda b, n: (0, 0)),
            pl.BlockSpec((E_D, N_E), lambda b, n: (0, 0)),
            pl.BlockSpec((N_E, E_D), lambda b, n: (0, 0)),
            pl.BlockSpec((1, N_E), lambda b, n: (0, 0)),
        ],
        out_specs=[
            pl.BlockSpec((1, C, R), lambda b, n: (b, 0, n)),
            pl.BlockSpec((1, 1), lambda b, n: (0, 0)),
            pl.BlockSpec((1, 1), lambda b, n: (0, 0)),
        ],
        out_shape=[
            jax.ShapeDtypeStruct((B, C, HW), jnp.float32),
            jax.ShapeDtypeStruct((1, 1), jnp.float32),
            jax.ShapeDtypeStruct((1, 1), jnp.float32),
        ],
        scratch_shapes=[pltpu.VMEM((1, N_E), jnp.float32)],
        compiler_params=pltpu.CompilerParams(
            dimension_semantics=("arbitrary", "arbitrary")),
    )(zr, cbt, cbt.astype(jnp.bfloat16), codebook, cn)


def kernel(z, codebook):
    B, C, H, W = z.shape
    zr = z.reshape(B, C, H * W)
    zq, loss, usage = _vq_tc(zr, codebook)
    return (zq.reshape(B, C, H, W), loss[0, 0], usage[0, 0])


# speedup vs baseline: 1.6642x; 1.0907x over previous
import functools

import jax
import jax.numpy as jnp
from jax import lax
from jax.experimental import pallas as pl
from jax.experimental.pallas import tpu as pltpu

N_E = 512
E_D = 32
R = 2048


def _vq_body(z_ref, cbt_ref, cbtbf_ref, cb_ref, cn_ref,
             zq_ref, loss_ref, usage_ref, flags_ref):
    b = pl.program_id(0)
    n = pl.program_id(1)
    first = (b == 0) & (n == 0)
    last = ((b == pl.num_programs(0) - 1) & (n == pl.num_programs(1) - 1))

    zblk = z_ref[0]
    cb = cb_ref[...]
    cn = cn_ref[...]

    zt = jnp.transpose(zblk)
    zn = jnp.sum(zt * zt, axis=1, keepdims=True)
    zc = lax.dot_general(
        zt.astype(jnp.bfloat16), cbtbf_ref[...],
        (((1,), (0,)), ((), ())),
        preferred_element_type=jnp.float32)
    dists = (zn + cn) - 2.0 * zc

    m = jnp.min(dists, axis=1, keepdims=True)
    ii = lax.broadcasted_iota(jnp.int32, (1, N_E), 1)
    idx = jnp.min(jnp.where(dists == m, ii, N_E), axis=1, keepdims=True)

    onehot = jnp.where(ii == idx, 1.0, 0.0)
    zq = lax.dot_general(
        cb, onehot, (((0,), (1,)), ((), ())),
        preferred_element_type=jnp.float32,
        precision=lax.Precision.HIGHEST)

    diff = zq - zblk
    zq_ref[0] = zblk + diff

    fl_part = jnp.max(onehot, axis=0, keepdims=True)

    @pl.when(first)
    def _():
        loss_ref[...] = jnp.zeros_like(loss_ref)
        flags_ref[...] = fl_part

    @pl.when(jnp.logical_not(first))
    def _():
        flags_ref[...] = jnp.maximum(flags_ref[...], fl_part)

    loss_ref[...] += jnp.sum(diff * diff).reshape(1, 1)

    @pl.when(last)
    def _():
        loss_ref[...] = loss_ref[...] * (1.25 / (16 * 32 * 64 * 64))
        usage_ref[...] = (jnp.sum(flags_ref[...]) * (1.0 / N_E)).reshape(1, 1)


@jax.jit
def _vq_tc(zr, codebook):
    B, C, HW = zr.shape
    nb = HW // R
    grid = (B, nb)
    cbt = codebook.T
    cn = jnp.sum(codebook * codebook, axis=1)[None, :]
    return pl.pallas_call(
        _vq_body,
        grid=grid,
        in_specs=[
            pl.BlockSpec((1, C, R), lambda b, n: (b, 0, n)),
            pl.BlockSpec((E_D, N_E), lambda b, n: (0, 0)),
            pl.BlockSpec((E_D, N_E), lambda b, n: (0, 0)),
            pl.BlockSpec((N_E, E_D), lambda b, n: (0, 0)),
            pl.BlockSpec((1, N_E), lambda b, n: (0, 0)),
        ],
        out_specs=[
            pl.BlockSpec((1, C, R), lambda b, n: (b, 0, n)),
            pl.BlockSpec((1, 1), lambda b, n: (0, 0)),
            pl.BlockSpec((1, 1), lambda b, n: (0, 0)),
        ],
        out_shape=[
            jax.ShapeDtypeStruct((B, C, HW), jnp.float32),
            jax.ShapeDtypeStruct((1, 1), jnp.float32),
            jax.ShapeDtypeStruct((1, 1), jnp.float32),
        ],
        scratch_shapes=[pltpu.VMEM((1, N_E), jnp.float32)],
        compiler_params=pltpu.CompilerParams(
            dimension_semantics=("arbitrary", "arbitrary")),
    )(zr, cbt, cbt.astype(jnp.bfloat16), codebook, cn)


def kernel(z, codebook):
    B, C, H, W = z.shape
    zr = z.reshape(B, C, H * W)
    zq, loss, usage = _vq_tc(zr, codebook)
    return (zq.reshape(B, C, H, W), loss[0, 0], usage[0, 0])


# speedup vs baseline: 2.5307x; 1.5207x over previous
import functools

import jax
import jax.numpy as jnp
from jax import lax
from jax.experimental import pallas as pl
from jax.experimental.pallas import tpu as pltpu

N_E = 512
E_D = 32
R = 2048


def _vq_body(z_ref, cbtbf_ref, cbt3_ref, cn_ref,
             zq_ref, loss_ref, usage_ref, flags_ref):
    b = pl.program_id(0)
    n = pl.program_id(1)
    first = (b == 0) & (n == 0)
    last = ((b == pl.num_programs(0) - 1) & (n == pl.num_programs(1) - 1))

    zblk = z_ref[0]
    cn = cn_ref[...]

    zt = jnp.transpose(zblk)
    zn = jnp.sum(zt * zt, axis=1, keepdims=True)
    zc = lax.dot_general(
        zt.astype(jnp.bfloat16), cbtbf_ref[...],
        (((1,), (0,)), ((), ())),
        preferred_element_type=jnp.float32)
    dists = (zn + cn) - 2.0 * zc

    m = jnp.min(dists, axis=1, keepdims=True)
    ii = lax.broadcasted_iota(jnp.int32, (1, N_E), 1)
    idx = jnp.min(jnp.where(dists == m, ii, N_E), axis=1, keepdims=True)

    idxt = jnp.transpose(idx)
    iit = lax.broadcasted_iota(jnp.int32, (N_E, 1), 0)
    oht = jnp.where(iit == idxt, 1.0, 0.0)
    fl_part = jnp.max(oht, axis=1, keepdims=True)
    zq3 = lax.dot_general(
        cbt3_ref[...], oht.astype(jnp.bfloat16),
        (((1,), (0,)), ((), ())),
        preferred_element_type=jnp.float32)
    zq = (zq3[0:E_D] + zq3[E_D:2 * E_D]) + zq3[2 * E_D:3 * E_D]

    diff = zq - zblk
    zq_ref[0] = zblk + diff

    @pl.when(first)
    def _():
        loss_ref[...] = jnp.zeros_like(loss_ref)
        flags_ref[...] = fl_part

    @pl.when(jnp.logical_not(first))
    def _():
        flags_ref[...] = jnp.maximum(flags_ref[...], fl_part)

    loss_ref[...] += jnp.sum(diff * diff).reshape(1, 1)

    @pl.when(last)
    def _():
        loss_ref[...] = loss_ref[...] * (1.25 / (16 * 32 * 64 * 64))
        usage_ref[...] = (jnp.sum(flags_ref[...]) * (1.0 / N_E)).reshape(1, 1)


@jax.jit
def _vq_tc(zr, codebook):
    B, C, HW = zr.shape
    nb = HW // R
    grid = (B, nb)
    cbt = codebook.T
    cn = jnp.sum(codebook * codebook, axis=1)[None, :]
    c1 = cbt.astype(jnp.bfloat16)
    r1 = cbt - c1.astype(jnp.float32)
    c2 = r1.astype(jnp.bfloat16)
    c3 = (r1 - c2.astype(jnp.float32)).astype(jnp.bfloat16)
    cbt3 = jnp.concatenate([c1, c2, c3], axis=0)
    return pl.pallas_call(
        _vq_body,
        grid=grid,
        in_specs=[
            pl.BlockSpec((1, C, R), lambda b, n: (b, 0, n)),
            pl.BlockSpec((E_D, N_E), lambda b, n: (0, 0)),
            pl.BlockSpec((3 * E_D, N_E), lambda b, n: (0, 0)),
            pl.BlockSpec((1, N_E), lambda b, n: (0, 0)),
        ],
        out_specs=[
            pl.BlockSpec((1, C, R), lambda b, n: (b, 0, n)),
            pl.BlockSpec((1, 1), lambda b, n: (0, 0)),
            pl.BlockSpec((1, 1), lambda b, n: (0, 0)),
        ],
        out_shape=[
            jax.ShapeDtypeStruct((B, C, HW), jnp.float32),
            jax.ShapeDtypeStruct((1, 1), jnp.float32),
            jax.ShapeDtypeStruct((1, 1), jnp.float32),
        ],
        scratch_shapes=[pltpu.VMEM((N_E, 1), jnp.float32)],
        compiler_params=pltpu.CompilerParams(
            dimension_semantics=("arbitrary", "arbitrary")),
    )(zr, cbt.astype(jnp.bfloat16), cbt3, cn)


def kernel(z, codebook):
    B, C, H, W = z.shape
    zr = z.reshape(B, C, H * W)
    zq, loss, usage = _vq_tc(zr, codebook)
    return (zq.reshape(B, C, H, W), loss[0, 0], usage[0, 0])
